# baseline (speedup 1.0000x reference)
import dataclasses
import functools

import jax
import jax.numpy as jnp
from jax import lax
from jax.experimental import pallas as pl
from jax.experimental.pallas import tpu as pltpu
from jax.experimental.pallas import tpu_sc as plsc

N, E, D = 10000, 320000, 128
P = 10240
PAD_IDX = N
NC, NS = 2, 16
NT = NC * NS
EPT = 10240
EP = NT * EPT
CH = 128
NCHUNK = EPT // CH
RPT = P // NS

Q = 4
QS = P // Q
DUMP = QS
QROWS = QS + 8
GT = N
QMUL = 13108
NSB = 2
SEG = NT * NSB
EPS = EPT // NSB
KMAX = EP // CH + SEG + 16
SUPC = 16
MAXSUP = (EP // CH + SEG + SUPC * NS) // (SUPC * NS) + 1

BR = 1280
NBLK = P // BR

_mesh = plsc.VectorSubcoreMesh(core_axis_name="c", subcore_axis_name="s")
_cp = pltpu.CompilerParams()
if "needs_layout_passes" in pltpu.CompilerParams.__dataclass_fields__:
    _cp = dataclasses.replace(_cp, needs_layout_passes=False)


HR = P // D
HRT = 8

@functools.partial(
    pl.kernel,
    out_type=jax.ShapeDtypeStruct((NC, HR, D), jnp.float32),
    mesh=_mesh,
    compiler_params=_cp,
    scratch_types=[
        pltpu.VMEM((NCHUNK, CH), jnp.int32),
        pltpu.VMEM((HR,), jnp.int32),
        pltpu.VMEM((HR, D), jnp.float32),
        pltpu.VMEM_SHARED((HR, D), jnp.float32),
    ],
)
def _sc_degree(dst_hbm, zrows_hbm, idhr_hbm, deg_hbm, didx, idhr, hist, dacc):
    c = lax.axis_index("c")
    s = lax.axis_index("s")
    wid = c * NS + s
    pltpu.sync_copy(dst_hbm.at[wid], didx)
    pltpu.sync_copy(zrows_hbm.at[pl.ds(0, HR)], hist)
    pltpu.sync_copy(idhr_hbm, idhr)

    @pl.when(s < HR // HRT)
    def _():
        pltpu.sync_copy(zrows_hbm.at[pl.ds(0, HRT)],
                        dacc.at[pl.ds(s * HRT, HRT)])

    plsc.subcore_barrier()
    ones16 = jnp.ones((16,), jnp.float32)

    @pl.loop(0, NCHUNK)
    def _(k):
        @pl.loop(0, CH // 16)
        def _(j):
            idx16 = didx[k, pl.ds(j * 16, 16)]
            row16 = jax.lax.shift_right_logical(idx16, 7)
            col16 = jnp.bitwise_and(idx16, 127)
            plsc.addupdate_scatter(hist, [row16, col16], ones16)

    pltpu.sync_copy(hist, dacc.at[idhr], add=True)
    plsc.subcore_barrier()

    @pl.when(s < HR // HRT)
    def _():
        pltpu.sync_copy(dacc.at[pl.ds(s * HRT, HRT)],
                        deg_hbm.at[c].at[pl.ds(s * HRT, HRT)])


def _tc_plan_body(dst_ref, offs_ref, kq_ref):
    d = dst_ref[...]
    q = lax.shift_right_logical(d * QMUL, 25)
    cols = []
    kqs = []
    for qi in range(Q):
        cnt = jnp.sum((q == qi).astype(jnp.float32), axis=1, keepdims=True)
        chunks = jnp.floor((cnt + 127.0) * (1.0 / 128.0))
        ii = lax.broadcasted_iota(jnp.int32, (SEG, SEG), 0)
        jj = lax.broadcasted_iota(jnp.int32, (SEG, SEG), 1)
        ltri = (ii > jj).astype(jnp.float32)
        off = lax.dot_general(ltri, chunks, (((1,), (0,)), ((), ())),
                              preferred_element_type=jnp.float32)
        cols.append(off)
        kqs.append(jnp.sum(chunks, keepdims=True).reshape(1, 1))
    z = jnp.zeros((SEG, 128 - Q), jnp.float32)
    offs_ref[...] = jnp.concatenate(cols + [z], axis=1).astype(jnp.int32)
    zk = jnp.zeros((1, 128 - Q), jnp.float32)
    kq_ref[...] = jnp.concatenate(kqs + [zk], axis=1).astype(jnp.int32)


def _tc_plan(dst2):
    return pl.pallas_call(
        _tc_plan_body,
        grid=(1,),
        in_specs=[pl.BlockSpec((SEG, EPS), lambda i: (0, 0))],
        out_specs=[
            pl.BlockSpec((SEG, 128), lambda i: (0, 0)),
            pl.BlockSpec((1, 128), lambda i: (0, 0)),
        ],
        out_shape=[
            jax.ShapeDtypeStruct((SEG, 128), jnp.int32),
            jax.ShapeDtypeStruct((1, 128), jnp.int32),
        ],
    )(dst2)


@functools.partial(
    pl.kernel,
    out_type=[
        jax.ShapeDtypeStruct((Q, KMAX, CH), jnp.int32),
        jax.ShapeDtypeStruct((Q, KMAX, CH), jnp.int32),
    ],
    mesh=_mesh,
    compiler_params=_cp,
    scratch_types=[
        pltpu.VMEM((EPS // 16, 16), jnp.int32),
        pltpu.VMEM((EPS // 16, 16), jnp.int32),
        pltpu.VMEM((128,), jnp.int32),
        pltpu.VMEM((Q * EPS + 16,), jnp.int32),
        pltpu.VMEM((Q * EPS + 16,), jnp.int32),
    ],
)
def _sc_bin(src_hbm, dst_hbm, offs_hbm, gsrc_hbm, gdst_hbm,
            sbuf, dbuf, offs_v, locsrc, locdst):
    c = lax.axis_index("c")
    s = lax.axis_index("s")
    wid = c * NS + s
    i16 = lax.iota(jnp.int32, 16)
    dumps16 = jnp.zeros((16,), jnp.int32)
    dumpd16 = jnp.full((16,), DUMP, jnp.int32)

    for sub in range(NSB):
        seg = wid * NSB + sub
        pltpu.sync_copy(src_hbm.at[seg], sbuf)
        pltpu.sync_copy(dst_hbm.at[seg], dbuf)
        pltpu.sync_copy(offs_hbm.at[seg], offs_v)

        def grp(i, curs):
            s16 = sbuf[i, pl.ds(0, 16)]
            d16 = dbuf[i, pl.ds(0, 16)]
            q16 = lax.shift_right_logical(d16 * QMUL, 25)
            srcw = jnp.where(s16 == PAD_IDX, 0, s16)
            new = []
            for qi in range(Q):
                m = q16 == qi
                mi = m.astype(jnp.int32)
                pos = plsc.cumsum(mi) - mi + (curs[qi] + qi * EPS)
                pos = jnp.where(m, pos, Q * EPS + i16)
                dl = jnp.where(d16 == PAD_IDX, DUMP, d16 - qi * QS)
                plsc.store_scatter(locsrc, [pos], srcw, mask=m)
                plsc.store_scatter(locdst, [pos], dl, mask=m)
                pc = plsc.all_reduce_population_count(m)
                new.append(curs[qi] + lax.reduce_max(pc, (0,)))
            return tuple(new)

        curs = lax.fori_loop(0, EPS // 16, grp, (0, 0, 0, 0))

        for qi in range(Q):
            cq = curs[qi]
            end = (cq + 127) & ~127
            nfill = lax.shift_right_logical(end - cq + 15, 4)

            def fill(t, _):
                p = cq + t * 16 + i16
                pm = p < end
                pos = jnp.where(pm, qi * EPS + p, Q * EPS + i16)
                plsc.store_scatter(locsrc, [pos], dumps16, mask=pm)
                plsc.store_scatter(locdst, [pos], dumpd16, mask=pm)
                return 0

            lax.fori_loop(0, nfill, fill, 0)
            nch = lax.shift_right_logical(cq + 127, 7)
            off = offs_v[pl.ds(0, 16)][qi]

            def wout(k, _):
                pltpu.sync_copy(locsrc.at[pl.ds(qi * EPS + k * CH, CH)],
                                gsrc_hbm.at[qi].at[off + k])
                pltpu.sync_copy(locdst.at[pl.ds(qi * EPS + k * CH, CH)],
                                gdst_hbm.at[qi].at[off + k])
                return 0

            lax.fori_loop(0, nch, wout, 0)


@functools.partial(
    pl.kernel,
    out_type=jax.ShapeDtypeStruct((Q, QS, D), jnp.float32),
    mesh=_mesh,
    scratch_types=[
        pltpu.VMEM((SUPC, CH), jnp.int32),
        pltpu.VMEM((SUPC, CH), jnp.int32),
        pltpu.VMEM((128,), jnp.int32),
        pltpu.VMEM((CH, D), jnp.float32),
        pltpu.VMEM_SHARED((GT, D), jnp.float32),
        pltpu.VMEM_SHARED((QROWS, D), jnp.float32),
    ],
)
def _sc_aggregate(g_hbm, gsrc_hbm, gdst_hbm, kq_hbm, zq_hbm, y_hbm,
                  sidx, didx, kq_v, rows, gtab, yacc):
    c = lax.axis_index("c")
    s = lax.axis_index("s")
    @pl.when(s < 10)
    def _():
        pltpu.sync_copy(g_hbm.at[pl.ds(s * 640, 640)],
                        gtab.at[pl.ds(s * 640, 640)])

    @pl.when(s >= 10)
    def _():
        pltpu.sync_copy(g_hbm.at[pl.ds(6400 + (s - 10) * 600, 600)],
                        gtab.at[pl.ds(6400 + (s - 10) * 600, 600)])

    pltpu.sync_copy(kq_hbm.at[0], kq_v)

    for r in range(2):
        q = 2 * r + c
        @pl.when(s < NS - 1)
        def _():
            pltpu.sync_copy(zq_hbm.at[pl.ds(0, 160)],
                            yacc.at[pl.ds(s * 160, 160)])

        @pl.when(s == NS - 1)
        def _():
            pltpu.sync_copy(zq_hbm, yacc.at[pl.ds(2400, 168)])

        plsc.subcore_barrier()
        kvec = kq_v[pl.ds(0, 16)]
        kq = jnp.where(c == 0, kvec[2 * r], kvec[2 * r + 1])
        for t in range(MAXSUP):
            j = s + t * NS
            base = j * SUPC

            @pl.when(base < kq)
            def _():
                pltpu.sync_copy(gsrc_hbm.at[q, pl.ds(base, SUPC)], sidx)
                pltpu.sync_copy(gdst_hbm.at[q, pl.ds(base, SUPC)], didx)
                nkk = jnp.minimum(SUPC, kq - base)

                def chunk(kk, _):
                    pltpu.sync_copy(gtab.at[sidx.at[kk]], rows)
                    pltpu.sync_copy(rows, yacc.at[didx.at[kk]], add=True)
                    return 0

                lax.fori_loop(0, nkk, chunk, 0)

        plsc.subcore_barrier()
        pltpu.sync_copy(yacc.at[pl.ds(s * 160, 160)],
                        y_hbm.at[q].at[pl.ds(s * 160, 160)])


def _tc1_body(deg_ref, x_ref, w_ref, g_ref, dinv_ref):
    d = deg_ref[0] + deg_ref[1] + 1.0
    dinv = lax.rsqrt(d)
    h = lax.dot_general(x_ref[...], w_ref[...], (((1,), (1,)), ((), ())),
                        preferred_element_type=jnp.float32)
    g_ref[...] = h * dinv
    dinv_ref[...] = dinv


def _tc1(degp, xp, W0):
    return pl.pallas_call(
        _tc1_body,
        grid=(NBLK,),
        in_specs=[
            pl.BlockSpec((NC, BR, 1), lambda i: (0, i, 0)),
            pl.BlockSpec((BR, D), lambda i: (i, 0)),
            pl.BlockSpec((D, D), lambda i: (0, 0)),
        ],
        out_specs=[
            pl.BlockSpec((BR, D), lambda i: (i, 0)),
            pl.BlockSpec((BR, 1), lambda i: (i, 0)),
        ],
        out_shape=[
            jax.ShapeDtypeStruct((P, D), jnp.float32),
            jax.ShapeDtypeStruct((P, 1), jnp.float32),
        ],
    )(degp, xp, W0)


def _tc2_body(y_ref, g0_ref, dinv_ref, b0_ref, w1_ref, g1_ref):
    dinv = dinv_ref[...]
    out0 = dinv * (y_ref[...] + g0_ref[...]) + b0_ref[...]
    h1 = lax.dot_general(out0, w1_ref[...], (((1,), (1,)), ((), ())),
                         preferred_element_type=jnp.float32)
    g1_ref[...] = h1 * dinv


def _tc2(y0, g0, dinv, b0r, W1):
    return pl.pallas_call(
        _tc2_body,
        grid=(NBLK,),
        in_specs=[
            pl.BlockSpec((BR, D), lambda i: (i, 0)),
            pl.BlockSpec((BR, D), lambda i: (i, 0)),
            pl.BlockSpec((BR, 1), lambda i: (i, 0)),
            pl.BlockSpec((1, D), lambda i: (0, 0)),
            pl.BlockSpec((D, D), lambda i: (0, 0)),
        ],
        out_specs=pl.BlockSpec((BR, D), lambda i: (i, 0)),
        out_shape=jax.ShapeDtypeStruct((P, D), jnp.float32),
    )(y0, g0, dinv, b0r, W1)


def _tc3_body(y_ref, g1_ref, dinv_ref, b1_ref, wout_ref, bout_ref,
              wm1_ref, bm1_ref, wm2_ref, bm2_ref, vnw_ref,
              nodeout_ref, vn_ref, acc_ref):
    i = pl.program_id(0)
    dinv = dinv_ref[...]
    t = dinv * (y_ref[...] + g1_ref[...]) + b1_ref[...]
    r = jnp.maximum(t, 0.0)
    nodeout_ref[...] = lax.dot_general(
        r, wout_ref[...], (((1,), (1,)), ((), ())),
        preferred_element_type=jnp.float32) + bout_ref[...]
    rows = lax.broadcasted_iota(jnp.int32, (BR, 1), 0) + i * BR
    rm = jnp.where(rows < N, r, 0.0)
    psum = jnp.sum(rm, axis=0, keepdims=True)

    @pl.when(i == 0)
    def _():
        acc_ref[...] = jnp.zeros_like(acc_ref)

    acc_ref[...] += psum

    @pl.when(i == NBLK - 1)
    def _():
        v = acc_ref[...] + vnw_ref[...]
        v1 = jnp.maximum(
            lax.dot_general(v, wm1_ref[...], (((1,), (1,)), ((), ())),
                            preferred_element_type=jnp.float32)
            + bm1_ref[...], 0.0)
        v2 = jnp.maximum(
            lax.dot_general(v1, wm2_ref[...], (((1,), (1,)), ((), ())),
                            preferred_element_type=jnp.float32)
            + bm2_ref[...], 0.0)
        vn_ref[...] = v2


def _tc3(y1, g1, dinv, b1r, Wout, boutr, Wm1, bm1r, Wm2, bm2r, vn_w):
    return pl.pallas_call(
        _tc3_body,
        grid=(NBLK,),
        in_specs=[
            pl.BlockSpec((BR, D), lambda i: (i, 0)),
            pl.BlockSpec((BR, D), lambda i: (i, 0)),
            pl.BlockSpec((BR, 1), lambda i: (i, 0)),
            pl.BlockSpec((1, D), lambda i: (0, 0)),
            pl.BlockSpec((D, D), lambda i: (0, 0)),
            pl.BlockSpec((1, D), lambda i: (0, 0)),
            pl.BlockSpec((D, D), lambda i: (0, 0)),
            pl.BlockSpec((1, D), lambda i: (0, 0)),
            pl.BlockSpec((D, D), lambda i: (0, 0)),
            pl.BlockSpec((1, D), lambda i: (0, 0)),
            pl.BlockSpec((1, D), lambda i: (0, 0)),
        ],
        out_specs=[
            pl.BlockSpec((BR, D), lambda i: (i, 0)),
            pl.BlockSpec((1, D), lambda i: (0, 0)),
        ],
        out_shape=[
            jax.ShapeDtypeStruct((P, D), jnp.float32),
            jax.ShapeDtypeStruct((1, D), jnp.float32),
        ],
        scratch_shapes=[pltpu.VMEM((1, D), jnp.float32)],
    )(y1, g1, dinv, b1r, Wout, boutr, Wm1, bm1r, Wm2, bm2r, vn_w)


def kernel(x, edge_index, W0, b0, W1, b1, Wm1, bm1, Wm2, bm2, Wout, bout, vn_w):
    xp = jnp.pad(x, ((0, P - N), (0, 0)))
    pad = jnp.full((EP - E,), PAD_IDX, jnp.int32)
    srcf = jnp.concatenate([edge_index[0], pad])
    dstf = jnp.concatenate([edge_index[1], pad])
    dstp = dstf.reshape(NT, NCHUNK, CH)
    dst2 = dstf.reshape(SEG, EPS)
    zrows = jnp.zeros((RPT, D), jnp.float32)
    zq = jnp.zeros((168, D), jnp.float32)
    idhr = jnp.arange(HR, dtype=jnp.int32)
    b0r = b0.reshape(1, D)
    b1r = b1.reshape(1, D)
    bm1r = bm1.reshape(1, D)
    bm2r = bm2.reshape(1, D)
    boutr = bout.reshape(1, D)

    offs, kq = _tc_plan(dst2)
    gsrc, gdst = _sc_bin(srcf.reshape(SEG, EPS // 16, 16),
                         dstf.reshape(SEG, EPS // 16, 16), offs)
    degp = _sc_degree(dstp, zrows, idhr)
    g0, dinv = _tc1(degp.reshape(NC, P, 1), xp, W0)
    y0 = _sc_aggregate(g0, gsrc, gdst, kq, zq).reshape(P, D)
    g1 = _tc2(y0, g0, dinv, b0r, W1)
    y1 = _sc_aggregate(g1, gsrc, gdst, kq, zq).reshape(P, D)
    node_out, vn = _tc3(y1, g1, dinv, b1r, Wout, boutr, Wm1, bm1r, Wm2, bm2r,
                        vn_w)
    return node_out[:N], vn
